# Initial kernel scaffold; baseline (speedup 1.0000x reference)
import jax
import jax.numpy as jnp
from jax.experimental import pallas as pl


def kernel(x, w0, a0, b0, w1, a1, b1, w2, a2, b2, w3, a3, b3):
    raise NotImplementedError("write your pallas kernel here")


# speedup vs baseline: 1.6619x; 1.6619x over previous
import functools

import jax
import jax.numpy as jnp
from jax import lax
from jax.experimental import pallas as pl
from jax.experimental.pallas import tpu as pltpu
from jax.experimental.pallas import tpu_sc as plsc

_NC = 2
_NS = 16
_NW = _NC * _NS
_L = 16
_B = 512

_C0 = (0, 0, 0, 0, 0, 0, 0, 0, 1, 1, 1, 1, 1, 1, 1, 1)
_C1 = (0, 0, 1, 1, 0, 0, 1, 1, -1, -1, 0, 0, -1, -1, 0, 0)
_C2 = (0, 0, 0, 0, 1, 1, 1, 1, -1, -1, -1, -1, 0, 0, 0, 0)
_C3 = (0, 1, -1, 0, -1, 0, -2, -1, 1, 2, 0, 1, 0, 1, -1, 0)


@functools.lru_cache(maxsize=None)
def _make_layer(din, dout, k_chunk):
    n_w = dout // _NW
    n_chunks = n_w // k_chunk
    mesh = plsc.VectorSubcoreMesh(core_axis_name="c", subcore_axis_name="s")

    @functools.partial(
        pl.kernel, mesh=mesh,
        out_type=jax.ShapeDtypeStruct((dout, _B), jnp.float32),
        compiler_params=pltpu.CompilerParams(needs_layout_passes=False),
        scratch_types=[
            pltpu.VMEM((16, n_w), jnp.float32),
            pltpu.VMEM((n_w,), jnp.float32),
            pltpu.VMEM((n_w,), jnp.float32),
            pltpu.VMEM((n_w,), jnp.float32),
            pltpu.VMEM((n_w,), jnp.float32),
            pltpu.VMEM((k_chunk,), jnp.int32),
            pltpu.VMEM((k_chunk,), jnp.int32),
            pltpu.VMEM((k_chunk, _B), jnp.float32),
            pltpu.VMEM((k_chunk, _B), jnp.float32),
            pltpu.VMEM((k_chunk, _B), jnp.float32),
            pltpu.SemaphoreType.DMA,
            pltpu.SemaphoreType.DMA,
        ],
    )
    def layer(xt, wt, ia, ib, out, wv, t0, t1, t2, t3, iav, ibv, av, bv, ov,
              sema, semb):
        wid = lax.axis_index("s") * _NC + lax.axis_index("c")
        base = wid * n_w
        pltpu.sync_copy(wt.at[pl.ds(wid * 16, 16)], wv)

        def coef_body(g, carry):
            sl = pl.ds(g * _L, _L)
            rows = [wv[i, sl] for i in range(16)]
            m = rows[0]
            for r in rows[1:]:
                m = jnp.maximum(m, r)
            es = [jnp.exp(r - m) for r in rows]
            s = es[0]
            for e in es[1:]:
                s = s + e
            inv = 1.0 / s

            def mix(coefs):
                acc = None
                for c, e in zip(coefs, es):
                    if c == 0:
                        continue
                    term = e if c == 1 else (-e if c == -1 else c * e)
                    acc = term if acc is None else acc + term
                return acc * inv

            t0[sl] = mix(_C0)
            t1[sl] = mix(_C1)
            t2[sl] = mix(_C2)
            t3[sl] = mix(_C3)
            return carry

        lax.fori_loop(0, n_w // _L, coef_body, 0)

        def chunk_body(c, carry):
            cb = base + c * k_chunk
            pltpu.sync_copy(ia.at[pl.ds(cb, k_chunk)], iav)
            pltpu.sync_copy(ib.at[pl.ds(cb, k_chunk)], ibv)
            cpa = pltpu.async_copy(xt.at[iav], av, sema)
            cpb = pltpu.async_copy(xt.at[ibv], bv, semb)
            cpa.wait()
            cpb.wait()

            def neuron_body(j, carry2):
                jj = c * k_chunk + j
                idx = jnp.full((_L,), jj, dtype=jnp.int32)
                c0 = plsc.load_gather(t0, [idx])
                c1 = plsc.load_gather(t1, [idx])
                c2 = plsc.load_gather(t2, [idx])
                c3 = plsc.load_gather(t3, [idx])
                for v in range(_B // _L):
                    sl = pl.ds(v * _L, _L)
                    a = av[j, sl]
                    b = bv[j, sl]
                    ov[j, sl] = c0 + c1 * a + c2 * b + c3 * (a * b)
                return carry2

            lax.fori_loop(0, k_chunk, neuron_body, 0)
            pltpu.sync_copy(ov, out.at[pl.ds(cb, k_chunk)])
            return carry

        lax.fori_loop(0, n_chunks, chunk_body, 0)

    return layer


_DIMS = ((1024, 8192), (8192, 8192), (8192, 8192), (8192, 512))


def kernel(x, w0, a0, b0, w1, a1, b1, w2, a2, b2, w3, a3, b3):
    ws = (w0, w1, w2, w3)
    ias = (a0, a1, a2, a3)
    ibs = (b0, b1, b2, b3)
    h = x.T
    for i, (din, dout) in enumerate(_DIMS):
        n_w = dout // _NW
        k_chunk = min(64, n_w)
        layer = _make_layer(din, dout, k_chunk)
        wt = ws[i].T.reshape(16, _NW, n_w).transpose(1, 0, 2).reshape(_NW * 16, n_w)
        h = layer(h, wt, ias[i], ibs[i])
    return h.T


# speedup vs baseline: 2.2508x; 1.3543x over previous
import functools

import jax
import jax.numpy as jnp
from jax import lax
from jax.experimental import pallas as pl
from jax.experimental.pallas import tpu as pltpu
from jax.experimental.pallas import tpu_sc as plsc

_NC = 2
_NS = 16
_NW = _NC * _NS
_L = 16
_B = 512

_C0 = (0, 0, 0, 0, 0, 0, 0, 0, 1, 1, 1, 1, 1, 1, 1, 1)
_C1 = (0, 0, 1, 1, 0, 0, 1, 1, -1, -1, 0, 0, -1, -1, 0, 0)
_C2 = (0, 0, 0, 0, 1, 1, 1, 1, -1, -1, -1, -1, 0, 0, 0, 0)
_C3 = (0, 1, -1, 0, -1, 0, -2, -1, 1, 2, 0, 1, 0, 1, -1, 0)


@functools.lru_cache(maxsize=None)
def _make_layer(din, dout, k_chunk):
    n_w = dout // _NW
    n_chunks = n_w // k_chunk
    mesh = plsc.VectorSubcoreMesh(core_axis_name="c", subcore_axis_name="s")

    @functools.partial(
        pl.kernel, mesh=mesh,
        out_type=jax.ShapeDtypeStruct((dout, _B), jnp.float32),
        compiler_params=pltpu.CompilerParams(needs_layout_passes=False),
        scratch_types=[
            pltpu.VMEM((n_w * 16,), jnp.float32),
            pltpu.VMEM((n_w,), jnp.float32),
            pltpu.VMEM((n_w,), jnp.float32),
            pltpu.VMEM((n_w,), jnp.float32),
            pltpu.VMEM((n_w,), jnp.float32),
            pltpu.VMEM((n_w,), jnp.int32),
            pltpu.VMEM((n_w,), jnp.int32),
            pltpu.VMEM((k_chunk, _B), jnp.float32),
            pltpu.VMEM((k_chunk, _B), jnp.float32),
            pltpu.VMEM((k_chunk, _B), jnp.float32),
            pltpu.VMEM((k_chunk, _B), jnp.float32),
            pltpu.VMEM((k_chunk, _B), jnp.float32),
            pltpu.VMEM((k_chunk, _B), jnp.float32),
            pltpu.SemaphoreType.DMA,
            pltpu.SemaphoreType.DMA,
            pltpu.SemaphoreType.DMA,
            pltpu.SemaphoreType.DMA,
            pltpu.SemaphoreType.DMA,
            pltpu.SemaphoreType.DMA,
        ],
    )
    def layer(xt, w, ia, ib, out, wv, t0, t1, t2, t3, iav, ibv,
              av0, av1, bv0, bv1, ov0, ov1, sa0, sa1, sb0, sb1, so0, so1):
        wid = lax.axis_index("s") * _NC + lax.axis_index("c")
        base = wid * n_w
        pltpu.sync_copy(ia.at[pl.ds(base, n_w)], iav)
        pltpu.sync_copy(ib.at[pl.ds(base, n_w)], ibv)
        pltpu.sync_copy(w.at[pl.ds(base * 16, n_w * 16)], wv)

        abufs, bbufs, obufs = (av0, av1), (bv0, bv1), (ov0, ov1)
        asems, bsems, osems = (sa0, sa1), (sb0, sb1), (so0, so1)

        def issue_gather(c):
            p = c % 2
            sl = pl.ds(c * k_chunk, k_chunk)
            ha = pltpu.async_copy(xt.at[iav.at[sl]], abufs[p], asems[p])
            hb = pltpu.async_copy(xt.at[ibv.at[sl]], bbufs[p], bsems[p])
            return ha, hb

        pend = {0: issue_gather(0)}
        if n_chunks > 1:
            pend[1] = issue_gather(1)

        lane = jnp.arange(_L, dtype=jnp.int32)

        def coef_body(g, carry):
            idxr = (g * _L + lane) * 16
            rows = [plsc.load_gather(wv, [idxr + i]) for i in range(16)]
            m = rows[0]
            for r in rows[1:]:
                m = jnp.maximum(m, r)
            es = [jnp.exp(r - m) for r in rows]
            s = es[0]
            for e in es[1:]:
                s = s + e
            inv = 1.0 / s

            def mix(coefs):
                acc = None
                for cf, e in zip(coefs, es):
                    if cf == 0:
                        continue
                    term = e if cf == 1 else (-e if cf == -1 else cf * e)
                    acc = term if acc is None else acc + term
                return acc * inv

            sl = pl.ds(g * _L, _L)
            t0[sl] = mix(_C0)
            t1[sl] = mix(_C1)
            t2[sl] = mix(_C2)
            t3[sl] = mix(_C3)
            return carry

        lax.fori_loop(0, n_w // _L, coef_body, 0)

        owaits = {}
        for c in range(n_chunks):
            p = c % 2
            ha, hb = pend.pop(c)
            ha.wait()
            hb.wait()
            if c - 2 in owaits:
                owaits.pop(c - 2).wait()
            av, bv, ov = abufs[p], bbufs[p], obufs[p]

            def neuron_body(j, carry2, _c=c, _av=av, _bv=bv, _ov=ov):
                jj = _c * k_chunk + j
                idx = jnp.full((_L,), jj, dtype=jnp.int32)
                c0 = plsc.load_gather(t0, [idx])
                c1 = plsc.load_gather(t1, [idx])
                c2 = plsc.load_gather(t2, [idx])
                c3 = plsc.load_gather(t3, [idx])
                for v in range(_B // _L):
                    sl = pl.ds(v * _L, _L)
                    a = _av[j, sl]
                    b = _bv[j, sl]
                    _ov[j, sl] = (c0 + c1 * a) + (c2 + c3 * a) * b
                return carry2

            lax.fori_loop(0, k_chunk, neuron_body, 0)
            owaits[c] = pltpu.async_copy(
                ov, out.at[pl.ds(base + c * k_chunk, k_chunk)], osems[p])
            if c + 2 < n_chunks:
                pend[c + 2] = issue_gather(c + 2)
        for h in owaits.values():
            h.wait()

    return layer


_DIMS = ((1024, 8192), (8192, 8192), (8192, 8192), (8192, 512))


def kernel(x, w0, a0, b0, w1, a1, b1, w2, a2, b2, w3, a3, b3):
    ws = (w0, w1, w2, w3)
    ias = (a0, a1, a2, a3)
    ibs = (b0, b1, b2, b3)
    h = x.T
    for i, (din, dout) in enumerate(_DIMS):
        n_w = dout // _NW
        k_chunk = min(32, n_w)
        layer = _make_layer(din, dout, k_chunk)
        h = layer(h, ws[i].reshape(dout * 16), ias[i], ibs[i])
    return h.T


# speedup vs baseline: 2.8064x; 1.2469x over previous
import functools

import jax
import jax.numpy as jnp
from jax import lax
from jax.experimental import pallas as pl
from jax.experimental.pallas import tpu as pltpu
from jax.experimental.pallas import tpu_sc as plsc

_NC = 2
_NS = 16
_NW = _NC * _NS
_L = 16
_B = 512

_C0 = (0, 0, 0, 0, 0, 0, 0, 0, 1, 1, 1, 1, 1, 1, 1, 1)
_C1 = (0, 0, 1, 1, 0, 0, 1, 1, -1, -1, 0, 0, -1, -1, 0, 0)
_C2 = (0, 0, 0, 0, 1, 1, 1, 1, -1, -1, -1, -1, 0, 0, 0, 0)
_C3 = (0, 1, -1, 0, -1, 0, -2, -1, 1, 2, 0, 1, 0, 1, -1, 0)


@functools.lru_cache(maxsize=None)
def _make_layer(din, dout, k_chunk):
    n_w = dout // _NW
    n_chunks = n_w // k_chunk
    mesh = plsc.VectorSubcoreMesh(core_axis_name="c", subcore_axis_name="s")

    _B2 = _B // 2
    @functools.partial(
        pl.kernel, mesh=mesh,
        out_type=jax.ShapeDtypeStruct((dout, _B2), jnp.int32),
        compiler_params=pltpu.CompilerParams(needs_layout_passes=False),
        scratch_types=[
            pltpu.VMEM((n_w * 16,), jnp.float32),
            pltpu.VMEM((n_w,), jnp.float32),
            pltpu.VMEM((n_w,), jnp.float32),
            pltpu.VMEM((n_w,), jnp.float32),
            pltpu.VMEM((n_w,), jnp.float32),
            pltpu.VMEM((n_w,), jnp.int32),
            pltpu.VMEM((n_w,), jnp.int32),
            pltpu.VMEM((k_chunk, _B2), jnp.int32),
            pltpu.VMEM((k_chunk, _B2), jnp.int32),
            pltpu.VMEM((k_chunk, _B2), jnp.int32),
            pltpu.VMEM((k_chunk, _B2), jnp.int32),
            pltpu.VMEM((k_chunk, _B2), jnp.int32),
            pltpu.VMEM((k_chunk, _B2), jnp.int32),
            pltpu.SemaphoreType.DMA,
            pltpu.SemaphoreType.DMA,
            pltpu.SemaphoreType.DMA,
            pltpu.SemaphoreType.DMA,
            pltpu.SemaphoreType.DMA,
            pltpu.SemaphoreType.DMA,
        ],
    )
    def layer(xt, w, ia, ib, out, wv, t0, t1, t2, t3, iav, ibv,
              av0, av1, bv0, bv1, ov0, ov1, sa0, sa1, sb0, sb1, so0, so1):
        wid = lax.axis_index("s") * _NC + lax.axis_index("c")
        base = wid * n_w
        pltpu.sync_copy(ia.at[pl.ds(base, n_w)], iav)
        pltpu.sync_copy(ib.at[pl.ds(base, n_w)], ibv)
        pltpu.sync_copy(w.at[pl.ds(base * 16, n_w * 16)], wv)

        abufs, bbufs, obufs = (av0, av1), (bv0, bv1), (ov0, ov1)
        asems, bsems, osems = (sa0, sa1), (sb0, sb1), (so0, so1)

        def issue_gather(c):
            p = c % 2
            sl = pl.ds(c * k_chunk, k_chunk)
            ha = pltpu.async_copy(xt.at[iav.at[sl]], abufs[p], asems[p])
            hb = pltpu.async_copy(xt.at[ibv.at[sl]], bbufs[p], bsems[p])
            return ha, hb

        pend = {0: issue_gather(0)}
        if n_chunks > 1:
            pend[1] = issue_gather(1)

        lane = jnp.arange(_L, dtype=jnp.int32)

        def coef_body(g, carry):
            idxr = (g * _L + lane) * 16
            rows = [plsc.load_gather(wv, [idxr + i]) for i in range(16)]
            m = rows[0]
            for r in rows[1:]:
                m = jnp.maximum(m, r)
            es = [jnp.exp(r - m) for r in rows]
            s = es[0]
            for e in es[1:]:
                s = s + e
            inv = 1.0 / s

            def mix(coefs):
                acc = None
                for cf, e in zip(coefs, es):
                    if cf == 0:
                        continue
                    term = e if cf == 1 else (-e if cf == -1 else cf * e)
                    acc = term if acc is None else acc + term
                return acc * inv

            sl = pl.ds(g * _L, _L)
            t0[sl] = mix(_C0)
            t1[sl] = mix(_C1)
            t2[sl] = mix(_C2)
            t3[sl] = mix(_C3)
            return carry

        lax.fori_loop(0, n_w // _L, coef_body, 0)

        owaits = {}
        for c in range(n_chunks):
            p = c % 2
            ha, hb = pend.pop(c)
            ha.wait()
            hb.wait()
            if c - 2 in owaits:
                owaits.pop(c - 2).wait()
            av, bv, ov = abufs[p], bbufs[p], obufs[p]

            def neuron_body(j, carry2, _c=c, _av=av, _bv=bv, _ov=ov):
                jj = _c * k_chunk + j
                idx = jnp.full((_L,), jj, dtype=jnp.int32)
                c0f = plsc.load_gather(t0, [idx])
                c1f = plsc.load_gather(t1, [idx])
                c2f = plsc.load_gather(t2, [idx])
                c3f = plsc.load_gather(t3, [idx])
                fmt = plsc.PackFormat.INTERLEAVED
                c0 = plsc.pack(c0f, c0f, format=fmt)
                c1 = plsc.pack(c1f, c1f, format=fmt)
                c2 = plsc.pack(c2f, c2f, format=fmt)
                c3 = plsc.pack(c3f, c3f, format=fmt)
                for v in range(_B2 // _L):
                    sl = pl.ds(v * _L, _L)
                    a = plsc.bitcast(_av[j, sl], jnp.bfloat16)
                    b = plsc.bitcast(_bv[j, sl], jnp.bfloat16)
                    r = (c0 + c1 * a) + (c2 + c3 * a) * b
                    _ov[j, sl] = plsc.bitcast(r, jnp.int32)
                return carry2

            lax.fori_loop(0, k_chunk, neuron_body, 0)
            owaits[c] = pltpu.async_copy(
                ov, out.at[pl.ds(base + c * k_chunk, k_chunk)], osems[p])
            if c + 2 < n_chunks:
                pend[c + 2] = issue_gather(c + 2)
        for h in owaits.values():
            h.wait()

    return layer


_DIMS = ((1024, 8192), (8192, 8192), (8192, 8192), (8192, 512))


def kernel(x, w0, a0, b0, w1, a1, b1, w2, a2, b2, w3, a3, b3):
    ws = (w0, w1, w2, w3)
    ias = (a0, a1, a2, a3)
    ibs = (b0, b1, b2, b3)
    hb = x.T.astype(jnp.bfloat16).reshape(_DIMS[0][0], _B // 2, 2)
    h = lax.bitcast_convert_type(hb, jnp.int32)
    for i, (din, dout) in enumerate(_DIMS):
        n_w = dout // _NW
        k_chunk = min(64, n_w)
        layer = _make_layer(din, dout, k_chunk)
        h = layer(h, ws[i].reshape(dout * 16), ias[i], ibs[i])
    out = lax.bitcast_convert_type(h, jnp.bfloat16).reshape(_DIMS[-1][1], _B)
    return out.T.astype(jnp.float32)
